# baseline (speedup 1.0000x reference)
import functools

import jax
import jax.numpy as jnp
from jax import lax
from jax.experimental import pallas as pl
from jax.experimental.pallas import tpu as pltpu
from jax.experimental.pallas import tpu_sc as plsc

NN = 10000
EE = 320000
DD = 128

NC = 2
NS = 16
NW = NC * NS
CHUNK = 128
NCH = 80
EPW = CHUNK * NCH
EPAD = EPW * NW
RPT = 632
NACC = RPT * NS


def _sc_agg_body(with_deg, table, srcs, dsts, *refs):
    if with_deg:
        (out, deg_out, acc_sh, deg_sh, src0, src1, dst0, dst1,
         rows0, rows1, ones_v, zdeg_v, isem0, isem1, gsem0, gsem1) = refs
    else:
        (out, acc_sh, src0, src1, dst0, dst1,
         rows0, rows1, isem0, isem1, gsem0, gsem1) = refs
    c = lax.axis_index("c")
    s = lax.axis_index("s")
    wid = s * NC + c
    r0 = s * RPT
    e0 = wid * EPW

    def idx_issue(ch, sv, dv, sem):
        base = pl.multiple_of(e0 + ch * CHUNK, CHUNK)
        pltpu.async_copy(srcs.at[pl.ds(base, CHUNK)], sv, sem)
        pltpu.async_copy(dsts.at[pl.ds(base, CHUNK)], dv, sem)

    def idx_wait(sv, dv, sem):
        pltpu.make_async_copy(srcs.at[pl.ds(0, CHUNK)], sv, sem).wait()
        pltpu.make_async_copy(dsts.at[pl.ds(0, CHUNK)], dv, sem).wait()

    def zrows(i, _):
        rows0[i // 8, pl.ds((i % 8) * 16, 16)] = jnp.zeros((16,), jnp.float32)
        return 0
    lax.fori_loop(0, CHUNK * 8, zrows, 0)

    if with_deg:
        def zdeg(i, _):
            zdeg_v[i] = jnp.zeros((16,), jnp.float32)
            return 0
        lax.fori_loop(0, CHUNK, zdeg, 0)

        def fones(i, _):
            ones_v[i] = jnp.ones((16,), jnp.float32)
            return 0
        lax.fori_loop(0, CHUNK, fones, 0)

    idx_issue(0, src0, dst0, isem0)
    idx_issue(1, src1, dst1, isem1)

    for r in range(4):
        pltpu.sync_copy(rows0, acc_sh.at[pl.ds(r0 + r * CHUNK, CHUNK)])
    pltpu.sync_copy(rows0.at[pl.ds(0, RPT - 4 * CHUNK)],
                    acc_sh.at[pl.ds(r0 + 4 * CHUNK, RPT - 4 * CHUNK)])
    if with_deg:
        for r in range(4):
            pltpu.sync_copy(zdeg_v, deg_sh.at[pl.ds(r0 + r * CHUNK, CHUNK)])
        pltpu.sync_copy(zdeg_v.at[pl.ds(0, RPT - 4 * CHUNK)],
                        deg_sh.at[pl.ds(r0 + 4 * CHUNK, RPT - 4 * CHUNK)])
    plsc.subcore_barrier()

    idx_wait(src0, dst0, isem0)
    pltpu.async_copy(table.at[src0], rows0, gsem0)

    def scat(rv, dv):
        pltpu.sync_copy(rv, acc_sh.at[dv], add=True)
        if with_deg:
            pltpu.sync_copy(ones_v, deg_sh.at[dv], add=True)

    def step(i, sv, dv, rv, gsem, nsv, ndv, nrv, nisem, ngsem, isem):
        pltpu.make_async_copy(table.at[sv], rv, gsem).wait()
        idx_wait(nsv, ndv, nisem)
        pltpu.async_copy(table.at[nsv], nrv, ngsem)
        scat(rv, dv)
        idx_issue(i + 2, sv, dv, isem)

    def group_body(g, _):
        step(2 * g, src0, dst0, rows0, gsem0,
             src1, dst1, rows1, isem1, gsem1, isem0)
        step(2 * g + 1, src1, dst1, rows1, gsem1,
             src0, dst0, rows0, isem0, gsem0, isem1)
        return 0
    lax.fori_loop(0, (NCH - 2) // 2, group_body, 0)

    pltpu.make_async_copy(table.at[src0], rows0, gsem0).wait()
    idx_wait(src1, dst1, isem1)
    pltpu.async_copy(table.at[src1], rows1, gsem1)
    scat(rows0, dst0)
    pltpu.make_async_copy(table.at[src1], rows1, gsem1).wait()
    scat(rows1, dst1)
    plsc.subcore_barrier()

    pltpu.sync_copy(acc_sh.at[pl.ds(r0, RPT)], out.at[c, pl.ds(r0, RPT)])
    if with_deg:
        pltpu.sync_copy(deg_sh.at[pl.ds(r0, RPT)], deg_out.at[c, pl.ds(r0, RPT)])


@functools.cache
def _sc_agg(with_deg):
    if with_deg:
        out_type = (
            jax.ShapeDtypeStruct((NC, NACC, DD), jnp.float32),
            jax.ShapeDtypeStruct((NC, NACC, 16), jnp.float32),
        )
        scratch = [
            pltpu.VMEM_SHARED((NACC, DD), jnp.float32),
            pltpu.VMEM_SHARED((NACC, 16), jnp.float32),
            pltpu.VMEM((CHUNK,), jnp.int32),
            pltpu.VMEM((CHUNK,), jnp.int32),
            pltpu.VMEM((CHUNK,), jnp.int32),
            pltpu.VMEM((CHUNK,), jnp.int32),
            pltpu.VMEM((CHUNK, DD), jnp.float32),
            pltpu.VMEM((CHUNK, DD), jnp.float32),
            pltpu.VMEM((CHUNK, 16), jnp.float32),
            pltpu.VMEM((CHUNK, 16), jnp.float32),
            pltpu.SemaphoreType.DMA,
            pltpu.SemaphoreType.DMA,
            pltpu.SemaphoreType.DMA,
            pltpu.SemaphoreType.DMA,
        ]
    else:
        out_type = jax.ShapeDtypeStruct((NC, NACC, DD), jnp.float32)
        scratch = [
            pltpu.VMEM_SHARED((NACC, DD), jnp.float32),
            pltpu.VMEM((CHUNK,), jnp.int32),
            pltpu.VMEM((CHUNK,), jnp.int32),
            pltpu.VMEM((CHUNK,), jnp.int32),
            pltpu.VMEM((CHUNK,), jnp.int32),
            pltpu.VMEM((CHUNK, DD), jnp.float32),
            pltpu.VMEM((CHUNK, DD), jnp.float32),
            pltpu.SemaphoreType.DMA,
            pltpu.SemaphoreType.DMA,
            pltpu.SemaphoreType.DMA,
            pltpu.SemaphoreType.DMA,
        ]
    return pl.kernel(
        functools.partial(_sc_agg_body, with_deg),
        out_type=out_type,
        mesh=plsc.VectorSubcoreMesh(core_axis_name="c", subcore_axis_name="s",
                                    num_cores=NC, num_subcores=NS),
        scratch_types=scratch,
        compiler_params=pltpu.CompilerParams(use_tc_tiling_on_sc=False),
    )


def _layer_body(h_ref, p0_ref, p1_ref, d0_ref, d1_ref,
                ws_ref, wn_ref, bs_ref, bn_ref, o_ref, *, relu):
    deg = jnp.maximum(d0_ref[:, 0:1] + d1_ref[:, 0:1], 1.0)
    agg = (p0_ref[...] + p1_ref[...]) / deg
    o = (jnp.dot(h_ref[...], ws_ref[...], preferred_element_type=jnp.float32)
         + jnp.dot(agg, wn_ref[...], preferred_element_type=jnp.float32)
         + bs_ref[...] + bn_ref[...])
    o_ref[...] = jnp.maximum(o, 0.0) if relu else o


def _sage_layer(h, p0, p1, d0, d1, ws, wn, bs, bn, relu):
    B = 1000
    grid = NN // B
    return pl.pallas_call(
        functools.partial(_layer_body, relu=relu),
        grid=(grid,),
        in_specs=[
            pl.BlockSpec((B, DD), lambda i: (i, 0)),
            pl.BlockSpec((B, DD), lambda i: (i, 0)),
            pl.BlockSpec((B, DD), lambda i: (i, 0)),
            pl.BlockSpec((B, 16), lambda i: (i, 0)),
            pl.BlockSpec((B, 16), lambda i: (i, 0)),
            pl.BlockSpec((DD, DD), lambda i: (0, 0)),
            pl.BlockSpec((DD, DD), lambda i: (0, 0)),
            pl.BlockSpec((1, DD), lambda i: (0, 0)),
            pl.BlockSpec((1, DD), lambda i: (0, 0)),
        ],
        out_specs=pl.BlockSpec((B, DD), lambda i: (i, 0)),
        out_shape=jax.ShapeDtypeStruct((NN, DD), jnp.float32),
    )(h, p0, p1, d0, d1, ws, wn, bs, bn)


def _dec_body(hb_ref, h_ref, wd_ref, bd_ref, o_ref):
    z = (jnp.dot(hb_ref[...], wd_ref[...], preferred_element_type=jnp.float32)
         + bd_ref[...])
    a = lax.dot_general(z, h_ref[...], (((1,), (1,)), ((), ())),
                        preferred_element_type=jnp.float32)
    a = jnp.maximum(a, 0.0)
    m = jnp.max(a, axis=1, keepdims=True)
    e = jnp.exp(a - m)
    o_ref[...] = e / jnp.sum(e, axis=1, keepdims=True)


def _decoder(h, wd, bd):
    B = 400
    grid = NN // B
    return pl.pallas_call(
        _dec_body,
        grid=(grid,),
        in_specs=[
            pl.BlockSpec((B, DD), lambda i: (i, 0)),
            pl.BlockSpec((NN, DD), lambda i: (0, 0)),
            pl.BlockSpec((DD, DD), lambda i: (0, 0)),
            pl.BlockSpec((1, DD), lambda i: (0, 0)),
        ],
        out_specs=pl.BlockSpec((B, NN), lambda i: (i, 0)),
        out_shape=jax.ShapeDtypeStruct((NN, NN), jnp.float32),
    )(h, h, wd, bd)


def kernel(inputs, edge_index, labels, W_self0, b_self0, W_neigh0, b_neigh0,
           W_self1, b_self1, W_neigh1, b_neigh1, W_dec, b_dec):
    npad = EPAD - EE
    src = jnp.concatenate([edge_index[0], jnp.zeros((npad,), jnp.int32)])
    dst = jnp.concatenate([edge_index[1], jnp.full((npad,), NN, jnp.int32)])

    parts0, degp = _sc_agg(True)(inputs, src, dst)
    d0 = degp[0, :NN]
    d1 = degp[1, :NN]
    h1 = _sage_layer(inputs, parts0[0, :NN], parts0[1, :NN], d0, d1,
                     W_self0, W_neigh0, b_self0.reshape(1, DD),
                     b_neigh0.reshape(1, DD), relu=True)

    parts1 = _sc_agg(False)(h1, src, dst)
    h2 = _sage_layer(h1, parts1[0, :NN], parts1[1, :NN], d0, d1,
                     W_self1, W_neigh1, b_self1.reshape(1, DD),
                     b_neigh1.reshape(1, DD), relu=False)

    adj = _decoder(h2, W_dec, b_dec.reshape(1, DD))
    return (adj, h2, labels)


# speedup vs baseline: 1.1276x; 1.1276x over previous
import functools

import jax
import jax.numpy as jnp
from jax import lax
from jax.experimental import pallas as pl
from jax.experimental.pallas import tpu as pltpu
from jax.experimental.pallas import tpu_sc as plsc

NN = 10000
EE = 320000
DD = 128

NC = 1
NS = 16
NW = NC * NS
CHUNK = 128
NCH = 158
EPW = CHUNK * NCH
EPAD = EPW * NW
RPT = 632
NACC = RPT * NS


def _sc_agg_body(with_deg, table, srcs, dsts, *refs):
    if with_deg:
        (out, deg_out, acc_sh, deg_sh, src0, src1, dst0, dst1,
         rows0, rows1, ones_v, zdeg_v, isem0, isem1, gsem0, gsem1) = refs
    else:
        (out, acc_sh, src0, src1, dst0, dst1,
         rows0, rows1, isem0, isem1, gsem0, gsem1) = refs
    c = lax.axis_index("c")
    s = lax.axis_index("s")
    wid = s * NC + c
    r0 = s * RPT
    e0 = wid * EPW

    def idx_issue(ch, sv, dv, sem):
        base = pl.multiple_of(e0 + ch * CHUNK, CHUNK)
        pltpu.async_copy(srcs.at[pl.ds(base, CHUNK)], sv, sem)
        pltpu.async_copy(dsts.at[pl.ds(base, CHUNK)], dv, sem)

    def idx_wait(sv, dv, sem):
        pltpu.make_async_copy(srcs.at[pl.ds(0, CHUNK)], sv, sem).wait()
        pltpu.make_async_copy(dsts.at[pl.ds(0, CHUNK)], dv, sem).wait()

    def zrows(i, _):
        rows0[i // 8, pl.ds((i % 8) * 16, 16)] = jnp.zeros((16,), jnp.float32)
        return 0
    lax.fori_loop(0, CHUNK * 8, zrows, 0)

    if with_deg:
        def zdeg(i, _):
            zdeg_v[i] = jnp.zeros((16,), jnp.float32)
            return 0
        lax.fori_loop(0, CHUNK, zdeg, 0)

        def fones(i, _):
            ones_v[i] = jnp.ones((16,), jnp.float32)
            return 0
        lax.fori_loop(0, CHUNK, fones, 0)

    idx_issue(0, src0, dst0, isem0)
    idx_issue(1, src1, dst1, isem1)

    for r in range(4):
        pltpu.sync_copy(rows0, acc_sh.at[pl.ds(r0 + r * CHUNK, CHUNK)])
    pltpu.sync_copy(rows0.at[pl.ds(0, RPT - 4 * CHUNK)],
                    acc_sh.at[pl.ds(r0 + 4 * CHUNK, RPT - 4 * CHUNK)])
    if with_deg:
        for r in range(4):
            pltpu.sync_copy(zdeg_v, deg_sh.at[pl.ds(r0 + r * CHUNK, CHUNK)])
        pltpu.sync_copy(zdeg_v.at[pl.ds(0, RPT - 4 * CHUNK)],
                        deg_sh.at[pl.ds(r0 + 4 * CHUNK, RPT - 4 * CHUNK)])
    plsc.subcore_barrier()

    idx_wait(src0, dst0, isem0)
    pltpu.async_copy(table.at[src0], rows0, gsem0)

    def scat(rv, dv):
        pltpu.sync_copy(rv, acc_sh.at[dv], add=True)
        if with_deg:
            pltpu.sync_copy(ones_v, deg_sh.at[dv], add=True)

    def step(i, sv, dv, rv, gsem, nsv, ndv, nrv, nisem, ngsem, isem):
        pltpu.make_async_copy(table.at[sv], rv, gsem).wait()
        idx_wait(nsv, ndv, nisem)
        pltpu.async_copy(table.at[nsv], nrv, ngsem)
        scat(rv, dv)
        idx_issue(i + 2, sv, dv, isem)

    def group_body(g, _):
        step(2 * g, src0, dst0, rows0, gsem0,
             src1, dst1, rows1, isem1, gsem1, isem0)
        step(2 * g + 1, src1, dst1, rows1, gsem1,
             src0, dst0, rows0, isem0, gsem0, isem1)
        return 0
    lax.fori_loop(0, (NCH - 2) // 2, group_body, 0)

    pltpu.make_async_copy(table.at[src0], rows0, gsem0).wait()
    idx_wait(src1, dst1, isem1)
    pltpu.async_copy(table.at[src1], rows1, gsem1)
    scat(rows0, dst0)
    pltpu.make_async_copy(table.at[src1], rows1, gsem1).wait()
    scat(rows1, dst1)
    plsc.subcore_barrier()

    pltpu.sync_copy(acc_sh.at[pl.ds(r0, RPT)], out.at[c, pl.ds(r0, RPT)])
    if with_deg:
        pltpu.sync_copy(deg_sh.at[pl.ds(r0, RPT)], deg_out.at[c, pl.ds(r0, RPT)])


@functools.cache
def _sc_agg(with_deg):
    if with_deg:
        out_type = (
            jax.ShapeDtypeStruct((NC, NACC, DD), jnp.float32),
            jax.ShapeDtypeStruct((NC, NACC, 16), jnp.float32),
        )
        scratch = [
            pltpu.VMEM_SHARED((NACC, DD), jnp.float32),
            pltpu.VMEM_SHARED((NACC, 16), jnp.float32),
            pltpu.VMEM((CHUNK,), jnp.int32),
            pltpu.VMEM((CHUNK,), jnp.int32),
            pltpu.VMEM((CHUNK,), jnp.int32),
            pltpu.VMEM((CHUNK,), jnp.int32),
            pltpu.VMEM((CHUNK, DD), jnp.float32),
            pltpu.VMEM((CHUNK, DD), jnp.float32),
            pltpu.VMEM((CHUNK, 16), jnp.float32),
            pltpu.VMEM((CHUNK, 16), jnp.float32),
            pltpu.SemaphoreType.DMA,
            pltpu.SemaphoreType.DMA,
            pltpu.SemaphoreType.DMA,
            pltpu.SemaphoreType.DMA,
        ]
    else:
        out_type = jax.ShapeDtypeStruct((NC, NACC, DD), jnp.float32)
        scratch = [
            pltpu.VMEM_SHARED((NACC, DD), jnp.float32),
            pltpu.VMEM((CHUNK,), jnp.int32),
            pltpu.VMEM((CHUNK,), jnp.int32),
            pltpu.VMEM((CHUNK,), jnp.int32),
            pltpu.VMEM((CHUNK,), jnp.int32),
            pltpu.VMEM((CHUNK, DD), jnp.float32),
            pltpu.VMEM((CHUNK, DD), jnp.float32),
            pltpu.SemaphoreType.DMA,
            pltpu.SemaphoreType.DMA,
            pltpu.SemaphoreType.DMA,
            pltpu.SemaphoreType.DMA,
        ]
    return pl.kernel(
        functools.partial(_sc_agg_body, with_deg),
        out_type=out_type,
        mesh=plsc.VectorSubcoreMesh(core_axis_name="c", subcore_axis_name="s",
                                    num_cores=NC, num_subcores=NS),
        scratch_types=scratch,
        compiler_params=pltpu.CompilerParams(use_tc_tiling_on_sc=False),
    )


def _layer_body(h_ref, p0_ref, d0_ref,
                ws_ref, wn_ref, bs_ref, bn_ref, o_ref, *, relu):
    deg = jnp.maximum(d0_ref[:, 0:1], 1.0)
    agg = p0_ref[...] / deg
    o = (jnp.dot(h_ref[...], ws_ref[...], preferred_element_type=jnp.float32)
         + jnp.dot(agg, wn_ref[...], preferred_element_type=jnp.float32)
         + bs_ref[...] + bn_ref[...])
    o_ref[...] = jnp.maximum(o, 0.0) if relu else o


def _sage_layer(h, p0, d0, ws, wn, bs, bn, relu):
    B = 1000
    grid = NN // B
    return pl.pallas_call(
        functools.partial(_layer_body, relu=relu),
        grid=(grid,),
        in_specs=[
            pl.BlockSpec((B, DD), lambda i: (i, 0)),
            pl.BlockSpec((B, DD), lambda i: (i, 0)),
            pl.BlockSpec((B, 16), lambda i: (i, 0)),
            pl.BlockSpec((DD, DD), lambda i: (0, 0)),
            pl.BlockSpec((DD, DD), lambda i: (0, 0)),
            pl.BlockSpec((1, DD), lambda i: (0, 0)),
            pl.BlockSpec((1, DD), lambda i: (0, 0)),
        ],
        out_specs=pl.BlockSpec((B, DD), lambda i: (i, 0)),
        out_shape=jax.ShapeDtypeStruct((NN, DD), jnp.float32),
    )(h, p0, d0, ws, wn, bs, bn)


def _dec_body(hb_ref, h_ref, wd_ref, bd_ref, o_ref):
    z = (jnp.dot(hb_ref[...], wd_ref[...], preferred_element_type=jnp.float32)
         + bd_ref[...])
    a = lax.dot_general(z, h_ref[...], (((1,), (1,)), ((), ())),
                        preferred_element_type=jnp.float32)
    a = jnp.maximum(a, 0.0)
    m = jnp.max(a, axis=1, keepdims=True)
    e = jnp.exp(a - m)
    o_ref[...] = e / jnp.sum(e, axis=1, keepdims=True)


def _decoder(h, wd, bd):
    B = 400
    grid = NN // B
    return pl.pallas_call(
        _dec_body,
        grid=(grid,),
        in_specs=[
            pl.BlockSpec((B, DD), lambda i: (i, 0)),
            pl.BlockSpec((NN, DD), lambda i: (0, 0)),
            pl.BlockSpec((DD, DD), lambda i: (0, 0)),
            pl.BlockSpec((1, DD), lambda i: (0, 0)),
        ],
        out_specs=pl.BlockSpec((B, NN), lambda i: (i, 0)),
        out_shape=jax.ShapeDtypeStruct((NN, NN), jnp.float32),
    )(h, h, wd, bd)


def kernel(inputs, edge_index, labels, W_self0, b_self0, W_neigh0, b_neigh0,
           W_self1, b_self1, W_neigh1, b_neigh1, W_dec, b_dec):
    npad = EPAD - EE
    src = jnp.concatenate([edge_index[0], jnp.zeros((npad,), jnp.int32)])
    dst = jnp.concatenate([edge_index[1], jnp.full((npad,), NN, jnp.int32)])

    parts0, degp = _sc_agg(True)(inputs, src, dst)
    d0 = degp[0, :NN]
    h1 = _sage_layer(inputs, parts0[0, :NN], d0,
                     W_self0, W_neigh0, b_self0.reshape(1, DD),
                     b_neigh0.reshape(1, DD), relu=True)

    parts1 = _sc_agg(False)(h1, src, dst)
    h2 = _sage_layer(h1, parts1[0, :NN], d0,
                     W_self1, W_neigh1, b_self1.reshape(1, DD),
                     b_neigh1.reshape(1, DD), relu=False)

    adj = _decoder(h2, W_dec, b_dec.reshape(1, DD))
    return (adj, h2, labels)
